# baseline (speedup 1.0000x reference)
import functools

import jax
import jax.numpy as jnp
from jax import lax
from jax.experimental import pallas as pl
from jax.experimental.pallas import tpu as pltpu
from jax.experimental.pallas import tpu_sc as plsc

D = 128
CHUNK = 64
NBUF = 5
LAG = 3


@functools.lru_cache(maxsize=None)
def _make_gather(B: int):
    info = plsc.get_sparse_core_info()
    nw = info.num_cores * info.num_subcores
    b_per_w = B // nw
    n_chunks = b_per_w // CHUNK
    n_groups = n_chunks // NBUF
    assert n_chunks % NBUF == 0
    mesh = plsc.VectorSubcoreMesh(core_axis_name="c", subcore_axis_name="s")

    @functools.partial(
        pl.kernel,
        mesh=mesh,
        out_type=jax.ShapeDtypeStruct((B, D), jnp.float32),
        scratch_types=(
            [pltpu.VMEM((n_chunks, CHUNK), jnp.int32)]
            + [pltpu.VMEM((CHUNK, D), jnp.float32) for _ in range(NBUF)]
            + [pltpu.SemaphoreType.DMA for _ in range(2 * NBUF)]
        ),
    )
    def k(idx_hbm, table_hbm, out_hbm, idx_v, *bufs_sems):
        rows = bufs_sems[:NBUF]
        gsem = bufs_sems[NBUF:2 * NBUF]
        ssem = bufs_sems[2 * NBUF:3 * NBUF]
        wid = lax.axis_index("s") * info.num_cores + lax.axis_index("c")
        pltpu.sync_copy(idx_hbm.at[wid], idx_v)
        base = wid * b_per_w

        def start_gather(c, b):
            pltpu.async_copy(table_hbm.at[idx_v.at[c]], rows[b], gsem[b])

        def wait_gather(c, b):
            pltpu.make_async_copy(
                table_hbm.at[idx_v.at[c]], rows[b], gsem[b]).wait()

        def start_scatter(c, b):
            pltpu.async_copy(
                rows[b], out_hbm.at[pl.ds(base + c * CHUNK, CHUNK)], ssem[b])

        def wait_scatter(c, b):
            pltpu.make_async_copy(
                rows[b], out_hbm.at[pl.ds(base + c * CHUNK, CHUNK)],
                ssem[b]).wait()

        for b in range(NBUF):
            start_gather(b, b)

        def group(g, carry):
            for b in range(NBUF):
                i = g * NBUF + b
                wait_gather(i, b)
                start_scatter(i, b)
                j = i - LAG
                bj = (b - LAG) % NBUF

                @pl.when(jnp.logical_and(j >= 0, j + NBUF < n_chunks))
                def _():
                    wait_scatter(j, bj)
                    start_gather(j + NBUF, bj)
            return carry

        lax.fori_loop(0, n_groups, group, 0)

        for c in range(n_chunks - NBUF, n_chunks):
            wait_scatter(c, c % NBUF)

    return k


def kernel(x, table):
    s0, s1 = x.shape
    B = s0 * s1
    info = plsc.get_sparse_core_info()
    nw = info.num_cores * info.num_subcores
    idx = x.reshape(nw, B // nw // CHUNK, CHUNK).astype(jnp.int32)
    out = _make_gather(B)(idx, table)
    return out.reshape(s0, s1, D)


# speedup vs baseline: 1.0573x; 1.0573x over previous
import functools

import jax
import jax.numpy as jnp
from jax import lax
from jax.experimental import pallas as pl
from jax.experimental.pallas import tpu as pltpu
from jax.experimental.pallas import tpu_sc as plsc

D = 128
CHUNK = 128
PAIR = 2 * CHUNK
NBUF = 3
LAG = 1


@functools.lru_cache(maxsize=None)
def _make_gather(B: int):
    info = plsc.get_sparse_core_info()
    nw = info.num_cores * info.num_subcores
    b_per_w = B // nw
    n_chunks = b_per_w // CHUNK
    n_pairs = n_chunks // 2
    n_groups = -(-n_pairs // NBUF)
    mesh = plsc.VectorSubcoreMesh(core_axis_name="c", subcore_axis_name="s")

    @functools.partial(
        pl.kernel,
        mesh=mesh,
        out_type=jax.ShapeDtypeStruct((B, D), jnp.float32),
        scratch_types=(
            [pltpu.VMEM((n_chunks, CHUNK), jnp.int32)]
            + [pltpu.VMEM((PAIR, D), jnp.float32) for _ in range(NBUF)]
            + [pltpu.SemaphoreType.DMA for _ in range(2 * NBUF)]
        ),
    )
    def k(idx_hbm, table_hbm, out_hbm, idx_v, *bufs_sems):
        rows = bufs_sems[:NBUF]
        gsem = bufs_sems[NBUF:2 * NBUF]
        ssem = bufs_sems[2 * NBUF:3 * NBUF]
        wid = lax.axis_index("s") * info.num_cores + lax.axis_index("c")
        pltpu.sync_copy(idx_hbm.at[wid], idx_v)
        base = wid * b_per_w

        def start_gathers(p, b):
            for h in range(2):
                pltpu.async_copy(table_hbm.at[idx_v.at[2 * p + h]],
                                 rows[b].at[pl.ds(h * CHUNK, CHUNK)], gsem[b])

        def wait_gathers(p, b):
            for h in range(2):
                pltpu.make_async_copy(
                    table_hbm.at[idx_v.at[2 * p + h]],
                    rows[b].at[pl.ds(h * CHUNK, CHUNK)], gsem[b]).wait()

        def start_scatter(p, b):
            pltpu.async_copy(
                rows[b], out_hbm.at[pl.ds(base + p * PAIR, PAIR)], ssem[b])

        def wait_scatter(p, b):
            pltpu.make_async_copy(
                rows[b], out_hbm.at[pl.ds(base + p * PAIR, PAIR)],
                ssem[b]).wait()

        for b in range(NBUF):
            start_gathers(b, b)

        def group(g, carry):
            for b in range(NBUF):
                p = g * NBUF + b

                @pl.when(p < n_pairs)
                def _():
                    wait_gathers(p, b)
                    start_scatter(p, b)

                j = p - LAG
                bj = (b - LAG) % NBUF

                @pl.when(jnp.logical_and(j >= 0, j + NBUF < n_pairs))
                def _():
                    wait_scatter(j, bj)
                    start_gathers(j + NBUF, bj)
            return carry

        lax.fori_loop(0, n_groups, group, 0)

        for p in range(n_pairs - NBUF, n_pairs):
            wait_scatter(p, p % NBUF)

    return k


def kernel(x, table):
    s0, s1 = x.shape
    B = s0 * s1
    info = plsc.get_sparse_core_info()
    nw = info.num_cores * info.num_subcores
    idx = x.reshape(nw, B // nw // CHUNK, CHUNK).astype(jnp.int32)
    out = _make_gather(B)(idx, table)
    return out.reshape(s0, s1, D)


# speedup vs baseline: 1.0718x; 1.0138x over previous
import functools

import jax
import jax.numpy as jnp
from jax import lax
from jax.experimental import pallas as pl
from jax.experimental.pallas import tpu as pltpu
from jax.experimental.pallas import tpu_sc as plsc

D = 128
CHUNK = 128
NBUF = 5
LAG = 3


@functools.lru_cache(maxsize=None)
def _make_gather(B: int):
    info = plsc.get_sparse_core_info()
    nw = info.num_cores * info.num_subcores
    b_per_w = B // nw
    n_chunks = b_per_w // CHUNK
    n_groups = n_chunks // NBUF
    assert n_chunks % NBUF == 0
    mesh = plsc.VectorSubcoreMesh(core_axis_name="c", subcore_axis_name="s")

    @functools.partial(
        pl.kernel,
        mesh=mesh,
        out_type=jax.ShapeDtypeStruct((B, D), jnp.float32),
        scratch_types=(
            [pltpu.VMEM((n_chunks, CHUNK), jnp.int32)]
            + [pltpu.VMEM((CHUNK, D), jnp.float32) for _ in range(NBUF)]
            + [pltpu.SemaphoreType.DMA for _ in range(2 * NBUF)]
        ),
    )
    def k(idx_hbm, table_hbm, out_hbm, idx_v, *bufs_sems):
        rows = bufs_sems[:NBUF]
        gsem = bufs_sems[NBUF:2 * NBUF]
        ssem = bufs_sems[2 * NBUF:3 * NBUF]
        wid = lax.axis_index("s") * info.num_cores + lax.axis_index("c")
        pltpu.sync_copy(idx_hbm.at[wid], idx_v)
        base = wid * b_per_w

        def start_gather(c, b):
            pltpu.async_copy(table_hbm.at[idx_v.at[c]], rows[b], gsem[b])

        def wait_gather(c, b):
            pltpu.make_async_copy(
                table_hbm.at[idx_v.at[c]], rows[b], gsem[b]).wait()

        def start_scatter(c, b):
            pltpu.async_copy(
                rows[b], out_hbm.at[pl.ds(base + c * CHUNK, CHUNK)], ssem[b])

        def wait_scatter(c, b):
            pltpu.make_async_copy(
                rows[b], out_hbm.at[pl.ds(base + c * CHUNK, CHUNK)],
                ssem[b]).wait()

        for b in range(NBUF):
            start_gather(b, b)

        def group(g, carry):
            for b in range(NBUF):
                i = g * NBUF + b
                wait_gather(i, b)
                start_scatter(i, b)
                j = i - LAG
                bj = (b - LAG) % NBUF

                @pl.when(jnp.logical_and(j >= 0, j + NBUF < n_chunks))
                def _():
                    wait_scatter(j, bj)
                    start_gather(j + NBUF, bj)
            return carry

        lax.fori_loop(0, n_groups, group, 0)

        for c in range(n_chunks - NBUF, n_chunks):
            wait_scatter(c, c % NBUF)

    return k


def kernel(x, table):
    s0, s1 = x.shape
    B = s0 * s1
    info = plsc.get_sparse_core_info()
    nw = info.num_cores * info.num_subcores
    idx = x.reshape(nw, B // nw // CHUNK, CHUNK).astype(jnp.int32)
    out = _make_gather(B)(idx, table)
    return out.reshape(s0, s1, D)
